# baseline (speedup 1.0000x reference)
import functools

import jax
import jax.numpy as jnp
from jax import lax
from jax.experimental import pallas as pl
from jax.experimental.pallas import tpu as pltpu
from jax.experimental.pallas import tpu_sc as plsc

_K = 1024
_D = 64
_DP = 128
_COST = 0.25
_N = 32 * 576
_NA = _N // 4
_NB = _N - _NA
_BN1 = 2304
_BN2 = 2304
_NW = 32
_BPW = _NA // _NW
_NCH = 2
_CH = _BPW // _NCH


def _argmin_body(x_ref, cbt2_ref, mean_ref, std_ref, idx_ref):
    x = x_ref[...]
    xn = (x - mean_ref[...]) / std_ref[...]
    cbt2 = cbt2_ref[...]
    dot2 = lax.dot_general(xn, cbt2, (((1,), (0,)), ((), ())),
                           preferred_element_type=jnp.float32)
    x2 = jnp.sum(xn * xn, axis=1, keepdims=True)
    c2 = 0.25 * jnp.sum(cbt2 * cbt2, axis=0, keepdims=True)
    d2 = (x2 + dot2) + c2
    dmin = jnp.min(d2, axis=1, keepdims=True)
    kiota = lax.broadcasted_iota(jnp.int32, d2.shape, 1).astype(jnp.float32)
    fidx = jnp.min(jnp.where(d2 == dmin, kiota, float(_K)), axis=1,
                   keepdims=True)
    idx_ref[...] = fidx.astype(jnp.int32)


def _fused_body(x_ref, cbt2_ref, mean_ref, std_ref, q_ref, idx_ref, loss_ref):
    x = x_ref[...]
    mean = mean_ref[...]
    std = std_ref[...]
    xn = (x - mean) / std
    cbt2 = cbt2_ref[...]
    dot2 = lax.dot_general(xn, cbt2, (((1,), (0,)), ((), ())),
                           preferred_element_type=jnp.float32)
    x2 = jnp.sum(xn * xn, axis=1, keepdims=True)
    c2 = 0.25 * jnp.sum(cbt2 * cbt2, axis=0, keepdims=True)
    d2 = (x2 + dot2) + c2
    dmin = jnp.min(d2, axis=1, keepdims=True)
    kiota = lax.broadcasted_iota(jnp.int32, d2.shape, 1).astype(jnp.float32)
    fidx = jnp.min(jnp.where(d2 == dmin, kiota, float(_K)), axis=1,
                   keepdims=True)
    idx_ref[...] = fidx.astype(jnp.int32)
    onehot = (kiota == fidx).astype(jnp.float32)
    qn2 = lax.dot_general(onehot, cbt2, (((1,), (1,)), ((), ())),
                          preferred_element_type=jnp.float32)
    q = qn2 * (-0.5 * std) + mean
    q_ref[...] = q
    loss_ref[...] = jnp.sum((x - q) ** 2).reshape(1, 1, 1)


_sc_mesh = plsc.VectorSubcoreMesh(core_axis_name="c", subcore_axis_name="s")


@functools.partial(
    pl.kernel,
    mesh=_sc_mesh,
    out_type=[
        jax.ShapeDtypeStruct((_NA * _D,), jnp.float32),
        jax.ShapeDtypeStruct((_NW * 16,), jnp.float32),
    ],
    scratch_types=[
        pltpu.VMEM((_BPW,), jnp.int32),
        pltpu.VMEM((_CH, _DP), jnp.float32),
        pltpu.VMEM((_CH, _DP), jnp.float32),
        pltpu.VMEM((_BPW * _D,), jnp.float32),
        pltpu.VMEM((_BPW * _D,), jnp.float32),
        pltpu.VMEM((_D,), jnp.float32),
        pltpu.VMEM((_D,), jnp.float32),
        pltpu.VMEM((16,), jnp.float32),
        pltpu.SemaphoreType.DMA,
        pltpu.SemaphoreType.DMA,
        pltpu.SemaphoreType.DMA,
    ],
)
def _sc_lookup(idx_hbm, cb_hbm, x_hbm, mean_hbm, std_hbm,
               q_hbm, loss_hbm,
               idx_v, gbuf0, gbuf1, x_v, out_v, mean_v, std_v, out16_v,
               sem0, sem1, sem_x):
    wid = lax.axis_index("s") * 2 + lax.axis_index("c")
    base = wid * _BPW
    gbufs = (gbuf0, gbuf1)
    sems = (sem0, sem1)
    pltpu.sync_copy(idx_hbm.at[pl.ds(base, _BPW)], idx_v)
    x_cp = pltpu.async_copy(x_hbm.at[pl.ds(base * _D, _BPW * _D)], x_v, sem_x)
    pltpu.sync_copy(mean_hbm, mean_v)
    pltpu.sync_copy(std_hbm, std_v)
    stats = [(std_v[pl.ds(16 * ci, 16)], mean_v[pl.ds(16 * ci, 16)])
             for ci in range(4)]

    copies = [pltpu.async_copy(
        cb_hbm.at[idx_v.at[pl.ds(0, _CH)]], gbufs[0], sems[0])]
    x_cp.wait()
    accs = (jnp.zeros((16,), jnp.float32),) * 4
    for ch in range(_NCH):
        copies[ch].wait()
        if ch + 1 < _NCH:
            copies.append(pltpu.async_copy(
                cb_hbm.at[idx_v.at[pl.ds((ch + 1) * _CH, _CH)]],
                gbufs[(ch + 1) % 2], sems[(ch + 1) % 2]))
        gbuf = gbufs[ch % 2]
        chbase = ch * _CH * _D

        def body(p, accs, gbuf=gbuf, chbase=chbase):
            new = list(accs)
            for r in range(2):
                row = 2 * p + r
                for ci in range(4):
                    fo = chbase + row * _D + ci * 16
                    q16 = (gbuf[row, pl.ds(ci * 16, 16)] * stats[ci][0]
                           + stats[ci][1])
                    out_v[pl.ds(fo, 16)] = q16
                    dd = x_v[pl.ds(fo, 16)] - q16
                    new[ci] = new[ci] + dd * dd
            return tuple(new)

        accs = lax.fori_loop(0, _CH // 2, body, accs)
    out16_v[...] = (accs[0] + accs[1]) + (accs[2] + accs[3])
    pltpu.sync_copy(out_v, q_hbm.at[pl.ds(base * _D, _BPW * _D)])
    pltpu.sync_copy(out16_v, loss_hbm.at[pl.ds(wid * 16, 16)])


def kernel(x, codebook, channel_means, channel_stds):
    B, T, D = x.shape
    N = B * T
    xf = x.reshape(N, D)
    cbt2 = codebook.T * (-2.0)
    cb_pad = jnp.concatenate(
        [codebook, jnp.zeros((_K, _DP - _D), jnp.float32)], axis=1)
    mean = channel_means.reshape(1, D)
    std = channel_stds.reshape(1, D)
    xa = xf[:_NA]
    xb = xf[_NA:]
    idx_a = pl.pallas_call(
        _argmin_body,
        grid=(_NA // _BN1,),
        in_specs=[
            pl.BlockSpec((_BN1, D), lambda i: (i, 0)),
            pl.BlockSpec((D, _K), lambda i: (0, 0)),
            pl.BlockSpec((1, D), lambda i: (0, 0)),
            pl.BlockSpec((1, D), lambda i: (0, 0)),
        ],
        out_specs=pl.BlockSpec((_BN1, 1), lambda i: (i, 0)),
        out_shape=jax.ShapeDtypeStruct((_NA, 1), jnp.int32),
    )(xa, cbt2, mean, std)
    q_a, loss_a = _sc_lookup(idx_a.reshape(_NA), cb_pad, xa.reshape(_NA * D),
                             channel_means, channel_stds)
    gb = _NB // _BN2
    q_b, idx_b, lp_b = pl.pallas_call(
        _fused_body,
        grid=(gb,),
        in_specs=[
            pl.BlockSpec((_BN2, D), lambda i: (i, 0)),
            pl.BlockSpec((D, _K), lambda i: (0, 0)),
            pl.BlockSpec((1, D), lambda i: (0, 0)),
            pl.BlockSpec((1, D), lambda i: (0, 0)),
        ],
        out_specs=[
            pl.BlockSpec((_BN2, D), lambda i: (i, 0)),
            pl.BlockSpec((_BN2, 1), lambda i: (i, 0)),
            pl.BlockSpec((1, 1, 1), lambda i: (i, 0, 0)),
        ],
        out_shape=[
            jax.ShapeDtypeStruct((_NB, D), jnp.float32),
            jax.ShapeDtypeStruct((_NB, 1), jnp.int32),
            jax.ShapeDtypeStruct((gb, 1, 1), jnp.float32),
        ],
    )(xb, cbt2, mean, std)
    quantized_st = jnp.concatenate(
        [q_a.reshape(_NA, D), q_b], axis=0).reshape(B, T, D)
    indices = jnp.concatenate([idx_a, idx_b], axis=0).reshape(B, T)
    loss = ((jnp.sum(loss_a) + jnp.sum(lp_b))
            * (_COST / (N * D)))
    return quantized_st, indices, loss


# speedup vs baseline: 1.6629x; 1.6629x over previous
import functools

import jax
import jax.numpy as jnp
from jax import lax
from jax.experimental import pallas as pl
from jax.experimental.pallas import tpu as pltpu

_K = 1024
_D = 64
_COST = 0.25
_BN = 4608


def _fused_body(x_ref, cbt2_ref, mean_ref, std_ref, q_ref, idx_ref, loss_ref):
    x = x_ref[...]
    mean = mean_ref[...]
    std = std_ref[...]
    xn = (x - mean) / std
    cbt2 = cbt2_ref[...]
    dot2 = lax.dot_general(xn, cbt2, (((1,), (0,)), ((), ())),
                           preferred_element_type=jnp.float32)
    x2 = jnp.sum(xn * xn, axis=1, keepdims=True)
    c2 = 0.25 * jnp.sum(cbt2 * cbt2, axis=0, keepdims=True)
    d2 = (x2 + dot2) + c2
    dmin = jnp.min(d2, axis=1, keepdims=True)
    kiota = lax.broadcasted_iota(jnp.int32, d2.shape, 1).astype(jnp.float32)
    fidx = jnp.min(jnp.where(d2 == dmin, kiota, float(_K)), axis=1,
                   keepdims=True)
    idx_ref[...] = fidx.astype(jnp.int32)
    onehot = (kiota == fidx).astype(jnp.float32)
    qn2 = lax.dot_general(onehot, cbt2, (((1,), (1,)), ((), ())),
                          preferred_element_type=jnp.float32)
    q = qn2 * (-0.5 * std) + mean
    q_ref[...] = q
    loss_ref[...] = jnp.sum((x - q) ** 2).reshape(1, 1, 1)


def kernel(x, codebook, channel_means, channel_stds):
    B, T, D = x.shape
    N = B * T
    G = N // _BN
    xf = x.reshape(N, D)
    cbt2 = codebook.T * (-2.0)
    mean = channel_means.reshape(1, D)
    std = channel_stds.reshape(1, D)
    q, idx, lp = pl.pallas_call(
        _fused_body,
        grid=(G,),
        in_specs=[
            pl.BlockSpec((_BN, D), lambda i: (i, 0)),
            pl.BlockSpec((D, _K), lambda i: (0, 0)),
            pl.BlockSpec((1, D), lambda i: (0, 0)),
            pl.BlockSpec((1, D), lambda i: (0, 0)),
        ],
        out_specs=[
            pl.BlockSpec((_BN, D), lambda i: (i, 0)),
            pl.BlockSpec((_BN, 1), lambda i: (i, 0)),
            pl.BlockSpec((1, 1, 1), lambda i: (i, 0, 0)),
        ],
        out_shape=[
            jax.ShapeDtypeStruct((N, D), jnp.float32),
            jax.ShapeDtypeStruct((N, 1), jnp.int32),
            jax.ShapeDtypeStruct((G, 1, 1), jnp.float32),
        ],
    )(xf, cbt2, mean, std)
    quantized_st = q.reshape(B, T, D)
    indices = idx.reshape(B, T)
    loss = jnp.sum(lp) * (_COST / (N * D))
    return quantized_st, indices, loss


# speedup vs baseline: 1.6643x; 1.0008x over previous
import functools

import jax
import jax.numpy as jnp
from jax import lax
from jax.experimental import pallas as pl
from jax.experimental.pallas import tpu as pltpu

_K = 1024
_D = 64
_COST = 0.25
_BN = 6144


def _fused_body(x_ref, cbt2_ref, mean_ref, std_ref, q_ref, idx_ref, loss_ref):
    x = x_ref[...]
    mean = mean_ref[...]
    std = std_ref[...]
    xn = (x - mean) / std
    cbt2 = cbt2_ref[...]
    dot2 = lax.dot_general(xn, cbt2, (((1,), (0,)), ((), ())),
                           preferred_element_type=jnp.float32)
    x2 = jnp.sum(xn * xn, axis=1, keepdims=True)
    c2 = 0.25 * jnp.sum(cbt2 * cbt2, axis=0, keepdims=True)
    d2 = (x2 + dot2) + c2
    dmin = jnp.min(d2, axis=1, keepdims=True)
    kiota = lax.broadcasted_iota(jnp.int32, d2.shape, 1).astype(jnp.float32)
    fidx = jnp.min(jnp.where(d2 == dmin, kiota, float(_K)), axis=1,
                   keepdims=True)
    idx_ref[...] = fidx.astype(jnp.int32)
    onehot = (kiota == fidx).astype(jnp.float32)
    qn2 = lax.dot_general(onehot, cbt2, (((1,), (1,)), ((), ())),
                          preferred_element_type=jnp.float32)
    q = qn2 * (-0.5 * std) + mean
    q_ref[...] = q
    loss_ref[...] = jnp.sum((x - q) ** 2).reshape(1, 1, 1)


def kernel(x, codebook, channel_means, channel_stds):
    B, T, D = x.shape
    N = B * T
    G = N // _BN
    xf = x.reshape(N, D)
    cbt2 = codebook.T * (-2.0)
    mean = channel_means.reshape(1, D)
    std = channel_stds.reshape(1, D)
    q, idx, lp = pl.pallas_call(
        _fused_body,
        grid=(G,),
        in_specs=[
            pl.BlockSpec((_BN, D), lambda i: (i, 0)),
            pl.BlockSpec((D, _K), lambda i: (0, 0)),
            pl.BlockSpec((1, D), lambda i: (0, 0)),
            pl.BlockSpec((1, D), lambda i: (0, 0)),
        ],
        out_specs=[
            pl.BlockSpec((_BN, D), lambda i: (i, 0)),
            pl.BlockSpec((_BN, 1), lambda i: (i, 0)),
            pl.BlockSpec((1, 1, 1), lambda i: (i, 0, 0)),
        ],
        out_shape=[
            jax.ShapeDtypeStruct((N, D), jnp.float32),
            jax.ShapeDtypeStruct((N, 1), jnp.int32),
            jax.ShapeDtypeStruct((G, 1, 1), jnp.float32),
        ],
    )(xf, cbt2, mean, std)
    quantized_st = q.reshape(B, T, D)
    indices = idx.reshape(B, T)
    loss = jnp.sum(lp) * (_COST / (N * D))
    return quantized_st, indices, loss


# speedup vs baseline: 1.6669x; 1.0016x over previous
import jax
import jax.numpy as jnp
from jax import lax
from jax.experimental import pallas as pl

_K = 1024
_D = 64
_COST = 0.25
_BN = 6144


def _fused_body(x_ref, cbt2_ref, mean_ref, std_ref, q_ref, idx_ref, loss_ref):
    x = x_ref[...]
    mean = mean_ref[...]
    std = std_ref[...]
    xn = (x - mean) / std
    cbt2 = cbt2_ref[...]
    dot2 = lax.dot_general(xn, cbt2, (((1,), (0,)), ((), ())),
                           preferred_element_type=jnp.float32)
    x2 = jnp.sum(xn * xn, axis=1, keepdims=True)
    c2 = 0.25 * jnp.sum(cbt2 * cbt2, axis=0, keepdims=True)
    d2 = (x2 + dot2) + c2
    dmin = jnp.min(d2, axis=1, keepdims=True)
    kiota = lax.broadcasted_iota(jnp.int32, d2.shape, 1).astype(jnp.float32)
    fidx = jnp.min(jnp.where(d2 == dmin, kiota, float(_K)), axis=1,
                   keepdims=True)
    idx_ref[...] = fidx.astype(jnp.int32)
    onehot = (kiota == fidx).astype(jnp.float32)
    qn2 = lax.dot_general(onehot, cbt2, (((1,), (1,)), ((), ())),
                          preferred_element_type=jnp.float32)
    q = qn2 * (-0.5 * std) + mean
    q_ref[...] = q
    loss_ref[...] = jnp.sum((x - q) ** 2).reshape(1, 1, 1)


def kernel(x, codebook, channel_means, channel_stds):
    B, T, D = x.shape
    N = B * T
    G = N // _BN
    xf = x.reshape(N, D)
    cbt2 = codebook.T * (-2.0)
    mean = channel_means.reshape(1, D)
    std = channel_stds.reshape(1, D)
    q, idx, lp = pl.pallas_call(
        _fused_body,
        grid=(G,),
        in_specs=[
            pl.BlockSpec((_BN, D), lambda i: (i, 0)),
            pl.BlockSpec((D, _K), lambda i: (0, 0)),
            pl.BlockSpec((1, D), lambda i: (0, 0)),
            pl.BlockSpec((1, D), lambda i: (0, 0)),
        ],
        out_specs=[
            pl.BlockSpec((_BN, D), lambda i: (i, 0)),
            pl.BlockSpec((_BN, 1), lambda i: (i, 0)),
            pl.BlockSpec((1, 1, 1), lambda i: (i, 0, 0)),
        ],
        out_shape=[
            jax.ShapeDtypeStruct((N, D), jnp.float32),
            jax.ShapeDtypeStruct((N, 1), jnp.int32),
            jax.ShapeDtypeStruct((G, 1, 1), jnp.float32),
        ],
    )(xf, cbt2, mean, std)
    quantized_st = q.reshape(B, T, D)
    indices = idx.reshape(B, T)
    loss = jnp.sum(lp) * (_COST / (N * D))
    return quantized_st, indices, loss
